# Initial kernel scaffold; baseline (speedup 1.0000x reference)
import jax
import jax.numpy as jnp
from jax.experimental import pallas as pl


def kernel(state, edge_index, edge_attr, W_gcn, b_gcn, bn_gamma, bn_beta, W_lin, b_lin):
    raise NotImplementedError("write your pallas kernel here")


# speedup vs baseline: 8.5614x; 8.5614x over previous
import functools

import jax
import jax.numpy as jnp
from jax import lax
from jax.experimental import pallas as pl
from jax.experimental.pallas import tpu as pltpu
from jax.experimental.pallas import tpu_sc as plsc

_N = 170000
_E = 160000
_D = 20
_H = 1024
_OUT = 2
_DPAD = 32

_RANGE = 5312
_RLAST = _N - 31 * _RANGE
_ACCR = _RLAST + 16
_DUMP = _RLAST

_K1_CHUNK = 8000
_K1_NCHUNK = _E // _K1_CHUNK


def _k1_body(dst_hbm, deg_hbm, dstv, hist):
    cid = lax.axis_index("c")
    sid = lax.axis_index("s")
    wid = sid * 2 + cid
    base = wid * _RANGE
    mylen = jnp.where(wid == 31, _RLAST, _RANGE)

    def zbody(i, _):
        hist[pl.ds(i * 16, 16)] = jnp.zeros((16,), jnp.float32)
        return 0
    lax.fori_loop(0, _RLAST // 16, zbody, 0)

    ones16 = jnp.full((16,), 1.0, jnp.float32)

    def cbody(k, _):
        pltpu.sync_copy(dst_hbm.at[pl.ds(k * _K1_CHUNK, _K1_CHUNK)], dstv)

        def ibody(j, _):
            d16 = dstv[pl.ds(j * 16, 16)]
            loc = d16 - base
            msk = (loc >= 0) & (loc < mylen)
            locc = jnp.where(msk, loc, 0)
            plsc.addupdate_scatter(hist, [locc], ones16, mask=msk)
            return 0
        lax.fori_loop(0, _K1_CHUNK // 16, ibody, 0)
        return 0
    lax.fori_loop(0, _K1_NCHUNK, cbody, 0)

    pltpu.sync_copy(hist.at[pl.ds(0, _RANGE)],
                    deg_hbm.at[pl.ds(base, _RANGE)])

    @pl.when(wid == 31)
    def _():
        pltpu.sync_copy(hist.at[pl.ds(_RANGE, _RLAST - _RANGE)],
                        deg_hbm.at[pl.ds(base + _RANGE, _RLAST - _RANGE)])


def _deg_sc(dst):
    mesh = plsc.VectorSubcoreMesh(core_axis_name="c", subcore_axis_name="s")
    fn = functools.partial(
        pl.kernel, mesh=mesh,
        out_type=jax.ShapeDtypeStruct((_N,), jnp.float32),
        scratch_types=[
            pltpu.VMEM((_K1_CHUNK,), jnp.int32),
            pltpu.VMEM((_RLAST,), jnp.float32),
        ],
        compiler_params=pltpu.CompilerParams(needs_layout_passes=False,
                                             use_tc_tiling_on_sc=False),
    )(_k1_body)
    return fn(dst)


_SEGE = 40000
_NSEG = _E // _SEGE
_CH = 2000
_NCH = _SEGE // _CH
_CB = 3072
_GC = _CB // 128


def _k3_body(src_hbm, dst_hbm, xs_hbm, z_hbm, srcv, dstv, cbs, cbd, gbuf,
             acc, sem):
    cid = lax.axis_index("c")
    sid = lax.axis_index("s")
    wid = sid * 2 + cid
    base = wid * _RANGE
    rng = jnp.where(wid == 31, _RLAST, _RANGE)
    iota16 = lax.iota(jnp.int32, 16)
    z16i = jnp.zeros((16,), jnp.int32)

    def zacc(i, _):
        acc[pl.ds(i * 16, 16)] = jnp.zeros((16,), jnp.float32)
        return 0
    lax.fori_loop(0, (_ACCR * _D) // 16, zacc, 0)

    def seg_body(s, _):
        def pre(i, _):
            cbs[pl.ds(i * 16, 16)] = z16i
            cbd[pl.ds(i * 16, 16)] = jnp.full((16,), _DUMP, jnp.int32)
            return 0
        lax.fori_loop(0, _CB // 16, pre, 0)

        def scan_chunk(k, cnt):
            off = s * _SEGE + k * _CH
            pltpu.sync_copy(src_hbm.at[pl.ds(off, _CH)], srcv)
            pltpu.sync_copy(dst_hbm.at[pl.ds(off, _CH)], dstv)

            def grp(j, cnt):
                d16 = dstv[pl.ds(j * 16, 16)]
                s16 = srcv[pl.ds(j * 16, 16)]
                loc = d16 - base
                msk = (loc >= 0) & (loc < rng)
                mi = jnp.where(msk, 1, 0)
                pos = cnt + plsc.cumsum(mi) - 1
                pos = jnp.minimum(pos, _CB - 1)
                plsc.store_scatter(cbs, [pos], s16, mask=msk)
                plsc.store_scatter(cbd, [pos], loc, mask=msk)
                return cnt + plsc.all_reduce_population_count(msk)
            return lax.fori_loop(0, _CH // 16, grp, cnt)

        cntv = lax.fori_loop(0, _NCH, scan_chunk, z16i)
        cnt = jnp.minimum(lax.reduce_max(cntv, (0,)), _CB)
        nch = (cnt + 127) // 128

        def gat_chunk(q, _):
            pltpu.async_copy(xs_hbm.at[cbs.at[pl.ds(q * 128, 128)]],
                             gbuf, sem).wait()
            for g in range(8):
                row16 = iota16 + g * 16
                ld16 = cbd[pl.ds(q * 128 + g * 16, 16)]
                ldb = ld16 * _D
                for c in range(_D):
                    vals = plsc.load_gather(
                        gbuf, [row16, jnp.full((16,), c, jnp.int32)])
                    plsc.addupdate_scatter(acc, [ldb + c], vals)
            return 0
        lax.fori_loop(0, nch, gat_chunk, 0)
        return 0
    lax.fori_loop(0, _NSEG, seg_body, 0)

    pltpu.sync_copy(acc.at[pl.ds(0, _RANGE * _D)],
                    z_hbm.at[pl.ds(base * _D, _RANGE * _D)])

    @pl.when(wid == 31)
    def _():
        pltpu.sync_copy(
            acc.at[pl.ds(_RANGE * _D, (_RLAST - _RANGE) * _D)],
            z_hbm.at[pl.ds(base * _D + _RANGE * _D,
                           (_RLAST - _RANGE) * _D)])


def _scatter_sc(src, dst, xs):
    mesh = plsc.VectorSubcoreMesh(core_axis_name="c", subcore_axis_name="s")
    fn = functools.partial(
        pl.kernel, mesh=mesh,
        out_type=jax.ShapeDtypeStruct((_N * _D,), jnp.float32),
        scratch_types=[
            pltpu.VMEM((_CH,), jnp.int32),
            pltpu.VMEM((_CH,), jnp.int32),
            pltpu.VMEM((_CB,), jnp.int32),
            pltpu.VMEM((_CB,), jnp.int32),
            pltpu.VMEM((128, _DPAD), jnp.float32),
            pltpu.VMEM((_ACCR * _D,), jnp.float32),
            pltpu.SemaphoreType.DMA,
        ],
        compiler_params=pltpu.CompilerParams(needs_layout_passes=False,
                                             use_tc_tiling_on_sc=False),
    )(_k3_body)
    return fn(src, dst, xs)


_BR = 10000
_NBLK = _N // _BR


def _k2_body(deg_ref, x_ref, xs_ref, dinv_ref):
    dinv = lax.rsqrt(deg_ref[...] + 1.0)
    dinv_ref[...] = dinv
    xs_ref[...] = jnp.concatenate(
        [x_ref[...] * dinv, jnp.zeros((_BR, _DPAD - _D), jnp.float32)],
        axis=1)


def _k4_body(z_ref, x_ref, dinv_ref, c_ref, s_ref):
    dinv = dinv_ref[...]
    y = dinv * z_ref[...] + (dinv * dinv) * x_ref[...]
    c = lax.dot_general(y, y, (((0,), (0,)), ((), ())),
                        preferred_element_type=jnp.float32)
    s = jnp.sum(y, axis=0, keepdims=True)

    @pl.when(pl.program_id(0) == 0)
    def _():
        c_ref[...] = c
        s_ref[...] = s

    @pl.when(pl.program_id(0) != 0)
    def _():
        c_ref[...] += c
        s_ref[...] += s


def _k5_body(c_ref, s_ref, w_ref, bg_ref, gam_ref, bet_ref, wl_ref, bl_ref,
             wc_ref, dv_ref):
    nn = jnp.float32(_N)
    w = w_ref[...]
    bg = bg_ref[...]
    sW = jnp.dot(s_ref[...], w, preferred_element_type=jnp.float32)
    mean = sW / nn + bg
    cw = jnp.dot(c_ref[...], w, preferred_element_type=jnp.float32)
    sumsq = jnp.sum(cw * w, axis=0, keepdims=True) \
        + 2.0 * bg * sW + nn * bg * bg
    var = sumsq / nn - mean * mean
    a = gam_ref[...] * lax.rsqrt(var + 1e-5)
    cvec = bet_ref[...] - mean * a
    wc_ref[...] = jnp.dot(w * a, wl_ref[...],
                          preferred_element_type=jnp.float32)
    dv_ref[...] = jnp.dot(bg * a + cvec, wl_ref[...],
                          preferred_element_type=jnp.float32) + bl_ref[...]


def _k6_body(z_ref, x_ref, dinv_ref, wc_ref, dv_ref, o_ref):
    dinv = dinv_ref[...]
    y = dinv * z_ref[...] + (dinv * dinv) * x_ref[...]
    p = jnp.dot(y, wc_ref[...], preferred_element_type=jnp.float32) \
        + dv_ref[...]
    p = jnp.maximum(p, 0.0)
    m = jnp.max(p, axis=1, keepdims=True)
    e = jnp.exp(p - m)
    o_ref[...] = e / jnp.sum(e, axis=1, keepdims=True)


def _row_spec(w):
    return pl.BlockSpec((_BR, w), lambda i: (i, 0))


def _full_spec(shape):
    return pl.BlockSpec(shape, lambda i: (0, 0))


def kernel(state, edge_index, edge_attr, W_gcn, b_gcn, bn_gamma, bn_beta,
           W_lin, b_lin):
    X = jnp.concatenate([state.reshape(-1, edge_attr.shape[1]), edge_attr],
                        axis=0)
    src = edge_index[0]
    dst = edge_index[1]

    deg = _deg_sc(dst)

    xs, dinv = pl.pallas_call(
        _k2_body,
        grid=(_NBLK,),
        in_specs=[_row_spec(1), _row_spec(_D)],
        out_specs=[_row_spec(_DPAD), _row_spec(1)],
        out_shape=[jax.ShapeDtypeStruct((_N, _DPAD), jnp.float32),
                   jax.ShapeDtypeStruct((_N, 1), jnp.float32)],
    )(deg.reshape(_N, 1), X)

    z = _scatter_sc(src, dst, xs).reshape(_N, _D)

    C, s = pl.pallas_call(
        _k4_body,
        grid=(_NBLK,),
        in_specs=[_row_spec(_D), _row_spec(_D), _row_spec(1)],
        out_specs=[_full_spec((_D, _D)), _full_spec((1, _D))],
        out_shape=[jax.ShapeDtypeStruct((_D, _D), jnp.float32),
                   jax.ShapeDtypeStruct((1, _D), jnp.float32)],
    )(z, X, dinv)

    Wc, dv = pl.pallas_call(
        _k5_body,
        out_shape=[jax.ShapeDtypeStruct((_D, _OUT), jnp.float32),
                   jax.ShapeDtypeStruct((1, _OUT), jnp.float32)],
    )(C, s, W_gcn, b_gcn.reshape(1, _H), bn_gamma.reshape(1, _H),
      bn_beta.reshape(1, _H), W_lin, b_lin.reshape(1, _OUT))

    out = pl.pallas_call(
        _k6_body,
        grid=(_NBLK,),
        in_specs=[_row_spec(_D), _row_spec(_D), _row_spec(1),
                  _full_spec((_D, _OUT)), _full_spec((1, _OUT))],
        out_specs=_row_spec(_OUT),
        out_shape=jax.ShapeDtypeStruct((_N, _OUT), jnp.float32),
    )(z, X, dinv, Wc, dv)

    return out


# speedup vs baseline: 9.4437x; 1.1031x over previous
import functools

import jax
import jax.numpy as jnp
from jax import lax
from jax.experimental import pallas as pl
from jax.experimental.pallas import tpu as pltpu
from jax.experimental.pallas import tpu_sc as plsc

_N = 170000
_E = 160000
_D = 20
_H = 1024
_OUT = 2
_DPAD = 32

_RANGE = 5312
_RLAST = _N - 31 * _RANGE
_ACCR = _RLAST + 16
_DUMP = _RLAST

_K1_CHUNK = 8000
_K1_NCHUNK = _E // _K1_CHUNK


def _k1_body(dst_hbm, deg_hbm, dstv, hist):
    cid = lax.axis_index("c")
    sid = lax.axis_index("s")
    wid = sid * 2 + cid
    base = wid * _RANGE
    mylen = jnp.where(wid == 31, _RLAST, _RANGE)

    def zbody(i, _):
        hist[pl.ds(i * 16, 16)] = jnp.zeros((16,), jnp.float32)
        return 0
    lax.fori_loop(0, _RLAST // 16, zbody, 0)

    ones16 = jnp.full((16,), 1.0, jnp.float32)

    def cbody(k, _):
        pltpu.sync_copy(dst_hbm.at[pl.ds(k * _K1_CHUNK, _K1_CHUNK)], dstv)

        def ibody(j, _):
            for u in range(5):
                d16 = dstv[pl.ds((j * 5 + u) * 16, 16)]
                loc = d16 - base
                msk = (loc >= 0) & (loc < mylen)
                locc = jnp.where(msk, loc, 0)
                plsc.addupdate_scatter(hist, [locc], ones16, mask=msk)
            return 0
        lax.fori_loop(0, _K1_CHUNK // 80, ibody, 0)
        return 0
    lax.fori_loop(0, _K1_NCHUNK, cbody, 0)

    pltpu.sync_copy(hist.at[pl.ds(0, _RANGE)],
                    deg_hbm.at[pl.ds(base, _RANGE)])

    @pl.when(wid == 31)
    def _():
        pltpu.sync_copy(hist.at[pl.ds(_RANGE, _RLAST - _RANGE)],
                        deg_hbm.at[pl.ds(base + _RANGE, _RLAST - _RANGE)])


def _deg_sc(dst):
    mesh = plsc.VectorSubcoreMesh(core_axis_name="c", subcore_axis_name="s")
    fn = functools.partial(
        pl.kernel, mesh=mesh,
        out_type=jax.ShapeDtypeStruct((_N,), jnp.float32),
        scratch_types=[
            pltpu.VMEM((_K1_CHUNK,), jnp.int32),
            pltpu.VMEM((_RLAST,), jnp.float32),
        ],
        compiler_params=pltpu.CompilerParams(needs_layout_passes=False,
                                             use_tc_tiling_on_sc=False),
    )(_k1_body)
    return fn(dst)


_SEGE = 40000
_NSEG = _E // _SEGE
_CH = 2000
_NCH = _SEGE // _CH
_CB = 3072
_GC = _CB // 128


def _k3_body(src_hbm, dst_hbm, xs_hbm, z_hbm, srcv, dstv, cbs, cbd, gbuf,
             acc, sem):
    cid = lax.axis_index("c")
    sid = lax.axis_index("s")
    wid = sid * 2 + cid
    base = wid * _RANGE
    rng = jnp.where(wid == 31, _RLAST, _RANGE)
    iota16 = lax.iota(jnp.int32, 16)
    z16i = jnp.zeros((16,), jnp.int32)

    def zacc(i, _):
        acc[pl.ds(i * 16, 16)] = jnp.zeros((16,), jnp.float32)
        return 0
    lax.fori_loop(0, (_ACCR * _D) // 16, zacc, 0)

    def seg_body(s, _):
        def pre(i, _):
            cbs[pl.ds(i * 16, 16)] = z16i
            cbd[pl.ds(i * 16, 16)] = jnp.full((16,), _DUMP, jnp.int32)
            return 0
        lax.fori_loop(0, _CB // 16, pre, 0)

        def scan_chunk(k, cnt):
            off = s * _SEGE + k * _CH
            pltpu.sync_copy(src_hbm.at[pl.ds(off, _CH)], srcv)
            pltpu.sync_copy(dst_hbm.at[pl.ds(off, _CH)], dstv)

            def grp(j, cnt):
                locs, msks, cums, pops, srcs = [], [], [], [], []
                for u in range(5):
                    d16 = dstv[pl.ds((j * 5 + u) * 16, 16)]
                    s16 = srcv[pl.ds((j * 5 + u) * 16, 16)]
                    loc = d16 - base
                    msk = (loc >= 0) & (loc < rng)
                    mi = jnp.where(msk, 1, 0)
                    locs.append(loc)
                    msks.append(msk)
                    srcs.append(s16)
                    cums.append(plsc.cumsum(mi))
                    pops.append(plsc.all_reduce_population_count(msk))
                for u in range(5):
                    pos = jnp.minimum(cnt + cums[u] - 1, _CB - 1)
                    plsc.store_scatter(cbs, [pos], srcs[u], mask=msks[u])
                    plsc.store_scatter(cbd, [pos], locs[u], mask=msks[u])
                    cnt = cnt + pops[u]
                return cnt
            return lax.fori_loop(0, _CH // 80, grp, cnt)

        cntv = lax.fori_loop(0, _NCH, scan_chunk, z16i)
        cnt = jnp.minimum(lax.reduce_max(cntv, (0,)), _CB)
        nch = (cnt + 127) // 128

        def gat_chunk(q, _):
            pltpu.async_copy(xs_hbm.at[cbs.at[pl.ds(q * 128, 128)]],
                             gbuf, sem).wait()
            for g in range(8):
                row16 = iota16 + g * 16
                ld16 = cbd[pl.ds(q * 128 + g * 16, 16)]
                ldb = ld16 * _D
                for c in range(_D):
                    vals = plsc.load_gather(
                        gbuf, [row16, jnp.full((16,), c, jnp.int32)])
                    plsc.addupdate_scatter(acc, [ldb + c], vals)
            return 0
        lax.fori_loop(0, nch, gat_chunk, 0)
        return 0
    lax.fori_loop(0, _NSEG, seg_body, 0)

    pltpu.sync_copy(acc.at[pl.ds(0, _RANGE * _D)],
                    z_hbm.at[pl.ds(base * _D, _RANGE * _D)])

    @pl.when(wid == 31)
    def _():
        pltpu.sync_copy(
            acc.at[pl.ds(_RANGE * _D, (_RLAST - _RANGE) * _D)],
            z_hbm.at[pl.ds(base * _D + _RANGE * _D,
                           (_RLAST - _RANGE) * _D)])


def _scatter_sc(src, dst, xs):
    mesh = plsc.VectorSubcoreMesh(core_axis_name="c", subcore_axis_name="s")
    fn = functools.partial(
        pl.kernel, mesh=mesh,
        out_type=jax.ShapeDtypeStruct((_N * _D,), jnp.float32),
        scratch_types=[
            pltpu.VMEM((_CH,), jnp.int32),
            pltpu.VMEM((_CH,), jnp.int32),
            pltpu.VMEM((_CB,), jnp.int32),
            pltpu.VMEM((_CB,), jnp.int32),
            pltpu.VMEM((128, _DPAD), jnp.float32),
            pltpu.VMEM((_ACCR * _D,), jnp.float32),
            pltpu.SemaphoreType.DMA,
        ],
        compiler_params=pltpu.CompilerParams(needs_layout_passes=False,
                                             use_tc_tiling_on_sc=False),
    )(_k3_body)
    return fn(src, dst, xs)


_BR = 10000
_NBLK = _N // _BR


def _k2_body(deg_ref, x_ref, xs_ref, dinv_ref):
    dinv = lax.rsqrt(deg_ref[...] + 1.0)
    dinv_ref[...] = dinv
    xs_ref[...] = jnp.concatenate(
        [x_ref[...] * dinv, jnp.zeros((_BR, _DPAD - _D), jnp.float32)],
        axis=1)


def _k4_body(z_ref, x_ref, dinv_ref, c_ref, s_ref):
    dinv = dinv_ref[...]
    y = dinv * z_ref[...] + (dinv * dinv) * x_ref[...]
    c = lax.dot_general(y, y, (((0,), (0,)), ((), ())),
                        preferred_element_type=jnp.float32)
    s = jnp.sum(y, axis=0, keepdims=True)

    @pl.when(pl.program_id(0) == 0)
    def _():
        c_ref[...] = c
        s_ref[...] = s

    @pl.when(pl.program_id(0) != 0)
    def _():
        c_ref[...] += c
        s_ref[...] += s


def _k5_body(c_ref, s_ref, w_ref, bg_ref, gam_ref, bet_ref, wl_ref, bl_ref,
             wc_ref, dv_ref):
    nn = jnp.float32(_N)
    w = w_ref[...]
    bg = bg_ref[...]
    sW = jnp.dot(s_ref[...], w, preferred_element_type=jnp.float32)
    mean = sW / nn + bg
    cw = jnp.dot(c_ref[...], w, preferred_element_type=jnp.float32)
    sumsq = jnp.sum(cw * w, axis=0, keepdims=True) \
        + 2.0 * bg * sW + nn * bg * bg
    var = sumsq / nn - mean * mean
    a = gam_ref[...] * lax.rsqrt(var + 1e-5)
    cvec = bet_ref[...] - mean * a
    wc_ref[...] = jnp.dot(w * a, wl_ref[...],
                          preferred_element_type=jnp.float32)
    dv_ref[...] = jnp.dot(bg * a + cvec, wl_ref[...],
                          preferred_element_type=jnp.float32) + bl_ref[...]


def _k6_body(z_ref, x_ref, dinv_ref, wc_ref, dv_ref, o_ref):
    dinv = dinv_ref[...]
    y = dinv * z_ref[...] + (dinv * dinv) * x_ref[...]
    p = jnp.dot(y, wc_ref[...], preferred_element_type=jnp.float32) \
        + dv_ref[...]
    p = jnp.maximum(p, 0.0)
    m = jnp.max(p, axis=1, keepdims=True)
    e = jnp.exp(p - m)
    o_ref[...] = e / jnp.sum(e, axis=1, keepdims=True)


def _row_spec(w):
    return pl.BlockSpec((_BR, w), lambda i: (i, 0))


def _full_spec(shape):
    return pl.BlockSpec(shape, lambda i: (0, 0))


def kernel(state, edge_index, edge_attr, W_gcn, b_gcn, bn_gamma, bn_beta,
           W_lin, b_lin):
    X = jnp.concatenate([state.reshape(-1, edge_attr.shape[1]), edge_attr],
                        axis=0)
    src = edge_index[0]
    dst = edge_index[1]

    deg = _deg_sc(dst)

    xs, dinv = pl.pallas_call(
        _k2_body,
        grid=(_NBLK,),
        in_specs=[_row_spec(1), _row_spec(_D)],
        out_specs=[_row_spec(_DPAD), _row_spec(1)],
        out_shape=[jax.ShapeDtypeStruct((_N, _DPAD), jnp.float32),
                   jax.ShapeDtypeStruct((_N, 1), jnp.float32)],
    )(deg.reshape(_N, 1), X)

    z = _scatter_sc(src, dst, xs).reshape(_N, _D)

    C, s = pl.pallas_call(
        _k4_body,
        grid=(_NBLK,),
        in_specs=[_row_spec(_D), _row_spec(_D), _row_spec(1)],
        out_specs=[_full_spec((_D, _D)), _full_spec((1, _D))],
        out_shape=[jax.ShapeDtypeStruct((_D, _D), jnp.float32),
                   jax.ShapeDtypeStruct((1, _D), jnp.float32)],
    )(z, X, dinv)

    Wc, dv = pl.pallas_call(
        _k5_body,
        out_shape=[jax.ShapeDtypeStruct((_D, _OUT), jnp.float32),
                   jax.ShapeDtypeStruct((1, _OUT), jnp.float32)],
    )(C, s, W_gcn, b_gcn.reshape(1, _H), bn_gamma.reshape(1, _H),
      bn_beta.reshape(1, _H), W_lin, b_lin.reshape(1, _OUT))

    out = pl.pallas_call(
        _k6_body,
        grid=(_NBLK,),
        in_specs=[_row_spec(_D), _row_spec(_D), _row_spec(1),
                  _full_spec((_D, _OUT)), _full_spec((1, _OUT))],
        out_specs=_row_spec(_OUT),
        out_shape=jax.ShapeDtypeStruct((_N, _OUT), jnp.float32),
    )(z, X, dinv, Wc, dv)

    return out


# speedup vs baseline: 9.8063x; 1.0384x over previous
import functools

import jax
import jax.numpy as jnp
from jax import lax
from jax.experimental import pallas as pl
from jax.experimental.pallas import tpu as pltpu
from jax.experimental.pallas import tpu_sc as plsc

_N = 170000
_E = 160000
_D = 20
_H = 1024
_OUT = 2
_DPAD = 32

_RANGE = 5312
_RLAST = _N - 31 * _RANGE
_ACCR = _RLAST + 16
_DUMP = _RLAST

_K1_CHUNK = 8000
_K1_NCHUNK = _E // _K1_CHUNK


def _k1_body(dst_hbm, deg_hbm, dstv, hist):
    cid = lax.axis_index("c")
    sid = lax.axis_index("s")
    wid = sid * 2 + cid
    base = wid * _RANGE
    mylen = jnp.where(wid == 31, _RLAST, _RANGE)

    def zbody(i, _):
        hist[pl.ds(i * 16, 16)] = jnp.zeros((16,), jnp.float32)
        return 0
    lax.fori_loop(0, _RLAST // 16, zbody, 0)

    ones16 = jnp.full((16,), 1.0, jnp.float32)

    def cbody(k, _):
        pltpu.sync_copy(dst_hbm.at[pl.ds(k * _K1_CHUNK, _K1_CHUNK)], dstv)

        def ibody(j, _):
            for u in range(5):
                d16 = dstv[pl.ds((j * 5 + u) * 16, 16)]
                loc = d16 - base
                msk = (loc >= 0) & (loc < mylen)
                locc = jnp.where(msk, loc, 0)
                plsc.addupdate_scatter(hist, [locc], ones16, mask=msk)
            return 0
        lax.fori_loop(0, _K1_CHUNK // 80, ibody, 0)
        return 0
    lax.fori_loop(0, _K1_NCHUNK, cbody, 0)

    pltpu.sync_copy(hist.at[pl.ds(0, _RANGE)],
                    deg_hbm.at[pl.ds(base, _RANGE)])

    @pl.when(wid == 31)
    def _():
        pltpu.sync_copy(hist.at[pl.ds(_RANGE, _RLAST - _RANGE)],
                        deg_hbm.at[pl.ds(base + _RANGE, _RLAST - _RANGE)])


def _deg_sc(dst):
    mesh = plsc.VectorSubcoreMesh(core_axis_name="c", subcore_axis_name="s")
    fn = functools.partial(
        pl.kernel, mesh=mesh,
        out_type=jax.ShapeDtypeStruct((_N,), jnp.float32),
        scratch_types=[
            pltpu.VMEM((_K1_CHUNK,), jnp.int32),
            pltpu.VMEM((_RLAST,), jnp.float32),
        ],
        compiler_params=pltpu.CompilerParams(needs_layout_passes=False,
                                             use_tc_tiling_on_sc=False),
    )(_k1_body)
    return fn(dst)


_SEGE = 40000
_NSEG = _E // _SEGE
_CH = 4000
_NCH = _SEGE // _CH
_CB = 3072
_GC = _CB // 128


def _k3_body(src_hbm, dst_hbm, xs_hbm, z_hbm, srcv, dstv, cbs, cbd, gbuf,
             acc, sem):
    cid = lax.axis_index("c")
    sid = lax.axis_index("s")
    wid = sid * 2 + cid
    base = wid * _RANGE
    rng = jnp.where(wid == 31, _RLAST, _RANGE)
    iota16 = lax.iota(jnp.int32, 16)
    z16i = jnp.zeros((16,), jnp.int32)

    def zacc(i, _):
        acc[pl.ds(i * 16, 16)] = jnp.zeros((16,), jnp.float32)
        return 0
    lax.fori_loop(0, (_ACCR * _D) // 16, zacc, 0)

    def seg_body(s, _):
        def pre(i, _):
            cbs[pl.ds(i * 16, 16)] = z16i
            cbd[pl.ds(i * 16, 16)] = jnp.full((16,), _DUMP, jnp.int32)
            return 0
        lax.fori_loop(0, _CB // 16, pre, 0)

        def scan_chunk(k, cnt):
            off = s * _SEGE + k * _CH
            pltpu.sync_copy(src_hbm.at[pl.ds(off, _CH)], srcv)
            pltpu.sync_copy(dst_hbm.at[pl.ds(off, _CH)], dstv)

            def grp(j, cnt):
                locs, msks, cums, pops, srcs = [], [], [], [], []
                for u in range(5):
                    d16 = dstv[pl.ds((j * 5 + u) * 16, 16)]
                    s16 = srcv[pl.ds((j * 5 + u) * 16, 16)]
                    loc = d16 - base
                    msk = (loc >= 0) & (loc < rng)
                    mi = jnp.where(msk, 1, 0)
                    locs.append(loc)
                    msks.append(msk)
                    srcs.append(s16)
                    cums.append(plsc.cumsum(mi))
                    pops.append(plsc.all_reduce_population_count(msk))
                for u in range(5):
                    pos = jnp.minimum(cnt + cums[u] - 1, _CB - 1)
                    plsc.store_scatter(cbs, [pos], srcs[u], mask=msks[u])
                    plsc.store_scatter(cbd, [pos], locs[u], mask=msks[u])
                    cnt = cnt + pops[u]
                return cnt
            return lax.fori_loop(0, _CH // 80, grp, cnt)

        cntv = lax.fori_loop(0, _NCH, scan_chunk, z16i)
        cnt = jnp.minimum(lax.reduce_max(cntv, (0,)), _CB)
        nch = (cnt + 127) // 128

        def gat_chunk(q, _):
            pltpu.async_copy(xs_hbm.at[cbs.at[pl.ds(q * 128, 128)]],
                             gbuf, sem).wait()
            for g in range(8):
                row16 = iota16 + g * 16
                ld16 = cbd[pl.ds(q * 128 + g * 16, 16)]
                ldb = ld16 * _D
                for c in range(_D):
                    vals = plsc.load_gather(
                        gbuf, [row16, jnp.full((16,), c, jnp.int32)])
                    plsc.addupdate_scatter(acc, [ldb + c], vals)
            return 0
        lax.fori_loop(0, nch, gat_chunk, 0)
        return 0
    lax.fori_loop(0, _NSEG, seg_body, 0)

    pltpu.sync_copy(acc.at[pl.ds(0, _RANGE * _D)],
                    z_hbm.at[pl.ds(base * _D, _RANGE * _D)])

    @pl.when(wid == 31)
    def _():
        pltpu.sync_copy(
            acc.at[pl.ds(_RANGE * _D, (_RLAST - _RANGE) * _D)],
            z_hbm.at[pl.ds(base * _D + _RANGE * _D,
                           (_RLAST - _RANGE) * _D)])


def _scatter_sc(src, dst, xs):
    mesh = plsc.VectorSubcoreMesh(core_axis_name="c", subcore_axis_name="s")
    fn = functools.partial(
        pl.kernel, mesh=mesh,
        out_type=jax.ShapeDtypeStruct((_N * _D,), jnp.float32),
        scratch_types=[
            pltpu.VMEM((_CH,), jnp.int32),
            pltpu.VMEM((_CH,), jnp.int32),
            pltpu.VMEM((_CB,), jnp.int32),
            pltpu.VMEM((_CB,), jnp.int32),
            pltpu.VMEM((128, _DPAD), jnp.float32),
            pltpu.VMEM((_ACCR * _D,), jnp.float32),
            pltpu.SemaphoreType.DMA,
        ],
        compiler_params=pltpu.CompilerParams(needs_layout_passes=False,
                                             use_tc_tiling_on_sc=False),
    )(_k3_body)
    return fn(src, dst, xs)


_BR = 10000
_NBLK = _N // _BR


def _k2_body(deg_ref, x_ref, xs_ref, dinv_ref):
    dinv = lax.rsqrt(deg_ref[...] + 1.0)
    dinv_ref[...] = dinv
    xs_ref[...] = jnp.concatenate(
        [x_ref[...] * dinv, jnp.zeros((_BR, _DPAD - _D), jnp.float32)],
        axis=1)


def _k4_body(z_ref, x_ref, dinv_ref, c_ref, s_ref):
    dinv = dinv_ref[...]
    y = dinv * z_ref[...] + (dinv * dinv) * x_ref[...]
    c = lax.dot_general(y, y, (((0,), (0,)), ((), ())),
                        preferred_element_type=jnp.float32)
    s = jnp.sum(y, axis=0, keepdims=True)

    @pl.when(pl.program_id(0) == 0)
    def _():
        c_ref[...] = c
        s_ref[...] = s

    @pl.when(pl.program_id(0) != 0)
    def _():
        c_ref[...] += c
        s_ref[...] += s


def _k5_body(c_ref, s_ref, w_ref, bg_ref, gam_ref, bet_ref, wl_ref, bl_ref,
             wc_ref, dv_ref):
    nn = jnp.float32(_N)
    w = w_ref[...]
    bg = bg_ref[...]
    sW = jnp.dot(s_ref[...], w, preferred_element_type=jnp.float32)
    mean = sW / nn + bg
    cw = jnp.dot(c_ref[...], w, preferred_element_type=jnp.float32)
    sumsq = jnp.sum(cw * w, axis=0, keepdims=True) \
        + 2.0 * bg * sW + nn * bg * bg
    var = sumsq / nn - mean * mean
    a = gam_ref[...] * lax.rsqrt(var + 1e-5)
    cvec = bet_ref[...] - mean * a
    wc_ref[...] = jnp.dot(w * a, wl_ref[...],
                          preferred_element_type=jnp.float32)
    dv_ref[...] = jnp.dot(bg * a + cvec, wl_ref[...],
                          preferred_element_type=jnp.float32) + bl_ref[...]


def _k6_body(z_ref, x_ref, dinv_ref, wc_ref, dv_ref, o_ref):
    dinv = dinv_ref[...]
    y = dinv * z_ref[...] + (dinv * dinv) * x_ref[...]
    p = jnp.dot(y, wc_ref[...], preferred_element_type=jnp.float32) \
        + dv_ref[...]
    p = jnp.maximum(p, 0.0)
    m = jnp.max(p, axis=1, keepdims=True)
    e = jnp.exp(p - m)
    o_ref[...] = e / jnp.sum(e, axis=1, keepdims=True)


def _row_spec(w):
    return pl.BlockSpec((_BR, w), lambda i: (i, 0))


def _full_spec(shape):
    return pl.BlockSpec(shape, lambda i: (0, 0))


def kernel(state, edge_index, edge_attr, W_gcn, b_gcn, bn_gamma, bn_beta,
           W_lin, b_lin):
    X = jnp.concatenate([state.reshape(-1, edge_attr.shape[1]), edge_attr],
                        axis=0)
    src = edge_index[0]
    dst = edge_index[1]

    deg = _deg_sc(dst)

    xs, dinv = pl.pallas_call(
        _k2_body,
        grid=(_NBLK,),
        in_specs=[_row_spec(1), _row_spec(_D)],
        out_specs=[_row_spec(_DPAD), _row_spec(1)],
        out_shape=[jax.ShapeDtypeStruct((_N, _DPAD), jnp.float32),
                   jax.ShapeDtypeStruct((_N, 1), jnp.float32)],
    )(deg.reshape(_N, 1), X)

    z = _scatter_sc(src, dst, xs).reshape(_N, _D)

    C, s = pl.pallas_call(
        _k4_body,
        grid=(_NBLK,),
        in_specs=[_row_spec(_D), _row_spec(_D), _row_spec(1)],
        out_specs=[_full_spec((_D, _D)), _full_spec((1, _D))],
        out_shape=[jax.ShapeDtypeStruct((_D, _D), jnp.float32),
                   jax.ShapeDtypeStruct((1, _D), jnp.float32)],
    )(z, X, dinv)

    Wc, dv = pl.pallas_call(
        _k5_body,
        out_shape=[jax.ShapeDtypeStruct((_D, _OUT), jnp.float32),
                   jax.ShapeDtypeStruct((1, _OUT), jnp.float32)],
    )(C, s, W_gcn, b_gcn.reshape(1, _H), bn_gamma.reshape(1, _H),
      bn_beta.reshape(1, _H), W_lin, b_lin.reshape(1, _OUT))

    out = pl.pallas_call(
        _k6_body,
        grid=(_NBLK,),
        in_specs=[_row_spec(_D), _row_spec(_D), _row_spec(1),
                  _full_spec((_D, _OUT)), _full_spec((1, _OUT))],
        out_specs=_row_spec(_OUT),
        out_shape=jax.ShapeDtypeStruct((_N, _OUT), jnp.float32),
    )(z, X, dinv, Wc, dv)

    return out


# speedup vs baseline: 9.8793x; 1.0074x over previous
import functools

import jax
import jax.numpy as jnp
from jax import lax
from jax.experimental import pallas as pl
from jax.experimental.pallas import tpu as pltpu
from jax.experimental.pallas import tpu_sc as plsc

_N = 170000
_E = 160000
_D = 20
_H = 1024
_OUT = 2
_DPAD = 32

_RANGE = 5312
_RLAST = _N - 31 * _RANGE
_ACCR = _RLAST + 16
_DUMP = _RLAST

_K1_CHUNK = 16000
_K1_NCHUNK = _E // _K1_CHUNK


def _k1_body(dst_hbm, deg_hbm, dstv, hist):
    cid = lax.axis_index("c")
    sid = lax.axis_index("s")
    wid = sid * 2 + cid
    base = wid * _RANGE
    mylen = jnp.where(wid == 31, _RLAST, _RANGE)

    def zbody(i, _):
        hist[pl.ds(i * 16, 16)] = jnp.zeros((16,), jnp.float32)
        return 0
    lax.fori_loop(0, _RLAST // 16, zbody, 0)

    ones16 = jnp.full((16,), 1.0, jnp.float32)

    def cbody(k, _):
        pltpu.sync_copy(dst_hbm.at[pl.ds(k * _K1_CHUNK, _K1_CHUNK)], dstv)

        def ibody(j, _):
            for u in range(5):
                d16 = dstv[pl.ds((j * 5 + u) * 16, 16)]
                loc = d16 - base
                msk = (loc >= 0) & (loc < mylen)
                locc = jnp.where(msk, loc, 0)
                plsc.addupdate_scatter(hist, [locc], ones16, mask=msk)
            return 0
        lax.fori_loop(0, _K1_CHUNK // 80, ibody, 0)
        return 0
    lax.fori_loop(0, _K1_NCHUNK, cbody, 0)

    pltpu.sync_copy(hist.at[pl.ds(0, _RANGE)],
                    deg_hbm.at[pl.ds(base, _RANGE)])

    @pl.when(wid == 31)
    def _():
        pltpu.sync_copy(hist.at[pl.ds(_RANGE, _RLAST - _RANGE)],
                        deg_hbm.at[pl.ds(base + _RANGE, _RLAST - _RANGE)])


def _deg_sc(dst):
    mesh = plsc.VectorSubcoreMesh(core_axis_name="c", subcore_axis_name="s")
    fn = functools.partial(
        pl.kernel, mesh=mesh,
        out_type=jax.ShapeDtypeStruct((_N,), jnp.float32),
        scratch_types=[
            pltpu.VMEM((_K1_CHUNK,), jnp.int32),
            pltpu.VMEM((_RLAST,), jnp.float32),
        ],
        compiler_params=pltpu.CompilerParams(needs_layout_passes=False,
                                             use_tc_tiling_on_sc=False),
    )(_k1_body)
    return fn(dst)


_SEGE = 40000
_NSEG = _E // _SEGE
_CH = 4000
_NCH = _SEGE // _CH
_CB = 3072
_GC = _CB // 128


def _k3_body(src_hbm, dst_hbm, xs_hbm, z_hbm, srcv, dstv, cbs, cbd, gbuf,
             acc, sem):
    cid = lax.axis_index("c")
    sid = lax.axis_index("s")
    wid = sid * 2 + cid
    base = wid * _RANGE
    rng = jnp.where(wid == 31, _RLAST, _RANGE)
    iota16 = lax.iota(jnp.int32, 16)
    z16i = jnp.zeros((16,), jnp.int32)

    def zacc(i, _):
        acc[pl.ds(i * 16, 16)] = jnp.zeros((16,), jnp.float32)
        return 0
    lax.fori_loop(0, (_ACCR * _D) // 16, zacc, 0)

    def seg_body(s, _):
        def pre(i, _):
            cbs[pl.ds(i * 16, 16)] = z16i
            cbd[pl.ds(i * 16, 16)] = jnp.full((16,), _DUMP, jnp.int32)
            return 0
        lax.fori_loop(0, _CB // 16, pre, 0)

        def scan_chunk(k, cnt):
            off = s * _SEGE + k * _CH
            pltpu.sync_copy(src_hbm.at[pl.ds(off, _CH)], srcv)
            pltpu.sync_copy(dst_hbm.at[pl.ds(off, _CH)], dstv)

            def grp(j, cnt):
                locs, msks, cums, pops, srcs = [], [], [], [], []
                for u in range(5):
                    d16 = dstv[pl.ds((j * 5 + u) * 16, 16)]
                    s16 = srcv[pl.ds((j * 5 + u) * 16, 16)]
                    loc = d16 - base
                    msk = (loc >= 0) & (loc < rng)
                    mi = jnp.where(msk, 1, 0)
                    locs.append(loc)
                    msks.append(msk)
                    srcs.append(s16)
                    cums.append(plsc.cumsum(mi))
                    pops.append(plsc.all_reduce_population_count(msk))
                for u in range(5):
                    pos = jnp.minimum(cnt + cums[u] - 1, _CB - 1)
                    plsc.store_scatter(cbs, [pos], srcs[u], mask=msks[u])
                    plsc.store_scatter(cbd, [pos], locs[u], mask=msks[u])
                    cnt = cnt + pops[u]
                return cnt
            return lax.fori_loop(0, _CH // 80, grp, cnt)

        cntv = lax.fori_loop(0, _NCH, scan_chunk, z16i)
        cnt = jnp.minimum(lax.reduce_max(cntv, (0,)), _CB)
        nch = (cnt + 127) // 128

        def gat_chunk(q, _):
            pltpu.async_copy(xs_hbm.at[cbs.at[pl.ds(q * 128, 128)]],
                             gbuf, sem).wait()
            for g in range(8):
                row16 = iota16 + g * 16
                ld16 = cbd[pl.ds(q * 128 + g * 16, 16)]
                ldb = ld16 * _D
                for c in range(_D):
                    vals = plsc.load_gather(
                        gbuf, [row16, jnp.full((16,), c, jnp.int32)])
                    plsc.addupdate_scatter(acc, [ldb + c], vals)
            return 0
        lax.fori_loop(0, nch, gat_chunk, 0)
        return 0
    lax.fori_loop(0, _NSEG, seg_body, 0)

    pltpu.sync_copy(acc.at[pl.ds(0, _RANGE * _D)],
                    z_hbm.at[pl.ds(base * _D, _RANGE * _D)])

    @pl.when(wid == 31)
    def _():
        pltpu.sync_copy(
            acc.at[pl.ds(_RANGE * _D, (_RLAST - _RANGE) * _D)],
            z_hbm.at[pl.ds(base * _D + _RANGE * _D,
                           (_RLAST - _RANGE) * _D)])


def _scatter_sc(src, dst, xs):
    mesh = plsc.VectorSubcoreMesh(core_axis_name="c", subcore_axis_name="s")
    fn = functools.partial(
        pl.kernel, mesh=mesh,
        out_type=jax.ShapeDtypeStruct((_N * _D,), jnp.float32),
        scratch_types=[
            pltpu.VMEM((_CH,), jnp.int32),
            pltpu.VMEM((_CH,), jnp.int32),
            pltpu.VMEM((_CB,), jnp.int32),
            pltpu.VMEM((_CB,), jnp.int32),
            pltpu.VMEM((128, _DPAD), jnp.float32),
            pltpu.VMEM((_ACCR * _D,), jnp.float32),
            pltpu.SemaphoreType.DMA,
        ],
        compiler_params=pltpu.CompilerParams(needs_layout_passes=False,
                                             use_tc_tiling_on_sc=False),
    )(_k3_body)
    return fn(src, dst, xs)


_BR = 10000
_NBLK = _N // _BR


def _k2_body(deg_ref, x_ref, xs_ref, dinv_ref):
    dinv = lax.rsqrt(deg_ref[...] + 1.0)
    dinv_ref[...] = dinv
    xs_ref[...] = jnp.concatenate(
        [x_ref[...] * dinv, jnp.zeros((_BR, _DPAD - _D), jnp.float32)],
        axis=1)


def _k4_body(z_ref, x_ref, dinv_ref, c_ref, s_ref):
    dinv = dinv_ref[...]
    y = dinv * z_ref[...] + (dinv * dinv) * x_ref[...]
    c = lax.dot_general(y, y, (((0,), (0,)), ((), ())),
                        preferred_element_type=jnp.float32)
    s = jnp.sum(y, axis=0, keepdims=True)

    @pl.when(pl.program_id(0) == 0)
    def _():
        c_ref[...] = c
        s_ref[...] = s

    @pl.when(pl.program_id(0) != 0)
    def _():
        c_ref[...] += c
        s_ref[...] += s


def _k5_body(c_ref, s_ref, w_ref, bg_ref, gam_ref, bet_ref, wl_ref, bl_ref,
             wc_ref, dv_ref):
    nn = jnp.float32(_N)
    w = w_ref[...]
    bg = bg_ref[...]
    sW = jnp.dot(s_ref[...], w, preferred_element_type=jnp.float32)
    mean = sW / nn + bg
    cw = jnp.dot(c_ref[...], w, preferred_element_type=jnp.float32)
    sumsq = jnp.sum(cw * w, axis=0, keepdims=True) \
        + 2.0 * bg * sW + nn * bg * bg
    var = sumsq / nn - mean * mean
    a = gam_ref[...] * lax.rsqrt(var + 1e-5)
    cvec = bet_ref[...] - mean * a
    wc_ref[...] = jnp.dot(w * a, wl_ref[...],
                          preferred_element_type=jnp.float32)
    dv_ref[...] = jnp.dot(bg * a + cvec, wl_ref[...],
                          preferred_element_type=jnp.float32) + bl_ref[...]


def _k6_body(z_ref, x_ref, dinv_ref, wc_ref, dv_ref, o_ref):
    dinv = dinv_ref[...]
    y = dinv * z_ref[...] + (dinv * dinv) * x_ref[...]
    p = jnp.dot(y, wc_ref[...], preferred_element_type=jnp.float32) \
        + dv_ref[...]
    p = jnp.maximum(p, 0.0)
    m = jnp.max(p, axis=1, keepdims=True)
    e = jnp.exp(p - m)
    o_ref[...] = e / jnp.sum(e, axis=1, keepdims=True)


def _row_spec(w):
    return pl.BlockSpec((_BR, w), lambda i: (i, 0))


def _full_spec(shape):
    return pl.BlockSpec(shape, lambda i: (0, 0))


def kernel(state, edge_index, edge_attr, W_gcn, b_gcn, bn_gamma, bn_beta,
           W_lin, b_lin):
    X = jnp.concatenate([state.reshape(-1, edge_attr.shape[1]), edge_attr],
                        axis=0)
    src = edge_index[0]
    dst = edge_index[1]

    deg = _deg_sc(dst)

    xs, dinv = pl.pallas_call(
        _k2_body,
        grid=(_NBLK,),
        in_specs=[_row_spec(1), _row_spec(_D)],
        out_specs=[_row_spec(_DPAD), _row_spec(1)],
        out_shape=[jax.ShapeDtypeStruct((_N, _DPAD), jnp.float32),
                   jax.ShapeDtypeStruct((_N, 1), jnp.float32)],
    )(deg.reshape(_N, 1), X)

    z = _scatter_sc(src, dst, xs).reshape(_N, _D)

    C, s = pl.pallas_call(
        _k4_body,
        grid=(_NBLK,),
        in_specs=[_row_spec(_D), _row_spec(_D), _row_spec(1)],
        out_specs=[_full_spec((_D, _D)), _full_spec((1, _D))],
        out_shape=[jax.ShapeDtypeStruct((_D, _D), jnp.float32),
                   jax.ShapeDtypeStruct((1, _D), jnp.float32)],
    )(z, X, dinv)

    Wc, dv = pl.pallas_call(
        _k5_body,
        out_shape=[jax.ShapeDtypeStruct((_D, _OUT), jnp.float32),
                   jax.ShapeDtypeStruct((1, _OUT), jnp.float32)],
    )(C, s, W_gcn, b_gcn.reshape(1, _H), bn_gamma.reshape(1, _H),
      bn_beta.reshape(1, _H), W_lin, b_lin.reshape(1, _OUT))

    out = pl.pallas_call(
        _k6_body,
        grid=(_NBLK,),
        in_specs=[_row_spec(_D), _row_spec(_D), _row_spec(1),
                  _full_spec((_D, _OUT)), _full_spec((1, _OUT))],
        out_specs=_row_spec(_OUT),
        out_shape=jax.ShapeDtypeStruct((_N, _OUT), jnp.float32),
    )(z, X, dinv, Wc, dv)

    return out
